# baseline (speedup 1.0000x reference)
import dataclasses
import functools

import jax
import jax.numpy as jnp
from jax import lax
from jax.experimental import pallas as pl
from jax.experimental.pallas import tpu as pltpu
from jax.experimental.pallas import tpu_sc as plsc

N = 10000
NP = 10240
E = 320000
HID = 128
HEADS = 8
HD = 16
NUM_CLASSES = 16
MSGW = 144

NC = 2
NS = 16
NW = NC * NS
EPW = E // NW
CHUNK = 40
NCHUNK = EPW // CHUNK
ROWS_PER_TILE = NP // NS
RCHUNK = 40
NRCHUNK = ROWS_PER_TILE // RCHUNK

_vec_mesh = plsc.VectorSubcoreMesh(core_axis_name="c", subcore_axis_name="s")

_sc_params = pltpu.CompilerParams(needs_layout_passes=False,
                                  use_tc_tiling_on_sc=False)


def _node_body(x_ref, wn_ref, bn_ref, wq_ref, bq_ref, wk_ref, bk_ref,
               wv_ref, bv_ref, ws_ref, bs_ref, qd_ref, kv_ref, skip_ref):
    h = jnp.dot(x_ref[...], wn_ref[...], preferred_element_type=jnp.float32) + bn_ref[...]
    q = jnp.dot(h, wq_ref[...], preferred_element_type=jnp.float32) + bq_ref[...]
    qd_ref[...] = q * 0.25
    kv_ref[:, :HID] = jnp.dot(h, wk_ref[...], preferred_element_type=jnp.float32) + bk_ref[...]
    kv_ref[:, HID:] = jnp.dot(h, wv_ref[...], preferred_element_type=jnp.float32) + bv_ref[...]
    skip_ref[...] = jnp.dot(h, ws_ref[...], preferred_element_type=jnp.float32) + bs_ref[...]


def _edge_emb_body(ea_ref, we_ref, be_ref, e_ref):
    e_ref[...] = jnp.dot(ea_ref[...], we_ref[...], preferred_element_type=jnp.float32) + be_ref[...]


def _sc_edge_pass(qd_hbm, kv_hbm, e_hbm, src_hbm, dst_hbm, part_hbm,
                  srcbuf, dstbuf, qbuf, kvbuf, ebuf, msgbuf, acc):
    cid = lax.axis_index("c")
    sid = lax.axis_index("s")
    wid = sid * NC + cid
    wbase = wid * EPW
    row0 = sid * ROWS_PER_TILE

    lane = lax.iota(jnp.int32, 16)

    @pl.loop(0, RCHUNK)
    def _zrow(i):
        for c in range(MSGW // 16):
            msgbuf[i, pl.ds(c * 16, 16)] = jnp.zeros((16,), jnp.float32)

    @pl.loop(0, NRCHUNK)
    def _zcopy(r):
        pltpu.sync_copy(msgbuf, acc.at[pl.ds(row0 + r * RCHUNK, RCHUNK)])

    plsc.subcore_barrier()

    @pl.loop(0, NCHUNK)
    def _chunk(c):
        base = pl.multiple_of(wbase + c * CHUNK, 8)
        pltpu.sync_copy(src_hbm.at[pl.ds(base, CHUNK)], srcbuf)
        pltpu.sync_copy(dst_hbm.at[pl.ds(base, CHUNK)], dstbuf)
        pltpu.sync_copy(qd_hbm.at[dstbuf], qbuf)
        pltpu.sync_copy(kv_hbm.at[srcbuf], kvbuf)
        pltpu.sync_copy(e_hbm.at[pl.ds(base, CHUNK)], ebuf)

        @pl.loop(0, CHUNK)
        def _edge(i):
            exrow = jnp.zeros((16,), jnp.float32)
            for h in range(HEADS):
                sl = pl.ds(h * HD, HD)
                qv = qbuf[i, sl]
                kvv = kvbuf[i, sl]
                ev = ebuf[i, sl]
                vv = kvbuf[i, pl.ds(HID + h * HD, HD)]
                s = jnp.sum(qv * (kvv + ev))
                exv = jnp.exp(jnp.full((16,), s, jnp.float32))
                msgbuf[i, sl] = exv * (vv + ev)
                exrow = jnp.where(lane == h, exv, exrow)
            msgbuf[i, pl.ds(HID, 16)] = exrow

        pltpu.sync_copy(msgbuf, acc.at[dstbuf], add=True)

    plsc.subcore_barrier()

    @pl.loop(0, NRCHUNK)
    def _dump(r):
        rr = pl.multiple_of(row0 + r * RCHUNK, 8)
        pltpu.sync_copy(acc.at[pl.ds(rr, RCHUNK)], msgbuf)
        pltpu.sync_copy(msgbuf, part_hbm.at[cid, pl.ds(rr, RCHUNK)])


def _combine_body(p0_ref, p1_ref, skip_ref, wo_ref, bo_ref, p_ref):
    agg = p0_ref[:, :HID] + p1_ref[:, :HID]
    den = p0_ref[:, HID:HID + 16] + p1_ref[:, HID:HID + 16]
    j_iota = lax.broadcasted_iota(jnp.int32, (16, HID), 0)
    c_iota = lax.broadcasted_iota(jnp.int32, (16, HID), 1)
    expand = (c_iota // HD == j_iota).astype(jnp.float32)
    den_exp = jnp.dot(den, expand, preferred_element_type=jnp.float32)
    h2 = agg / (den_exp + 1e-16) + skip_ref[...]
    p_ref[...] = jnp.dot(h2, wo_ref[...], preferred_element_type=jnp.float32) + bo_ref[...]


def _sc_readout(p_hbm, src_hbm, dst_hbm, out_hbm, srcbuf, dstbuf, abuf, bbuf):
    cid = lax.axis_index("c")
    sid = lax.axis_index("s")
    wid = sid * NC + cid
    wbase = wid * EPW

    @pl.loop(0, NCHUNK)
    def _chunk(c):
        base = pl.multiple_of(wbase + c * CHUNK, 8)
        pltpu.sync_copy(src_hbm.at[pl.ds(base, CHUNK)], srcbuf)
        pltpu.sync_copy(dst_hbm.at[pl.ds(base, CHUNK)], dstbuf)
        pltpu.sync_copy(p_hbm.at[srcbuf], abuf)
        pltpu.sync_copy(p_hbm.at[dstbuf], bbuf)

        @pl.loop(0, CHUNK)
        def _edge(i):
            abuf[i, :] = abuf[i, :] + bbuf[i, :]

        pltpu.sync_copy(abuf, out_hbm.at[pl.ds(base, CHUNK)])


def kernel(x, edge_index, edge_attr, W_node, b_node, W_edge, b_edge, W_q, b_q,
           W_k, b_k, W_v, b_v, W_skip, b_skip, W_out, b_out):
    src = edge_index[0]
    dst = edge_index[1]
    xp = jnp.pad(x, ((0, NP - N), (0, 0)))

    B1 = 1280
    qd, kv, skip = pl.pallas_call(
        _node_body,
        grid=(NP // B1,),
        in_specs=[pl.BlockSpec((B1, HID), lambda i: (i, 0))]
        + [pl.BlockSpec((HID, HID), lambda i: (0, 0)), pl.BlockSpec((1, HID), lambda i: (0, 0))] * 5,
        out_specs=[
            pl.BlockSpec((B1, HID), lambda i: (i, 0)),
            pl.BlockSpec((B1, 2 * HID), lambda i: (i, 0)),
            pl.BlockSpec((B1, HID), lambda i: (i, 0)),
        ],
        out_shape=[
            jax.ShapeDtypeStruct((NP, HID), jnp.float32),
            jax.ShapeDtypeStruct((NP, 2 * HID), jnp.float32),
            jax.ShapeDtypeStruct((NP, HID), jnp.float32),
        ],
    )(xp, W_node.T, b_node[None, :], W_q.T, b_q[None, :], W_k.T, b_k[None, :],
      W_v.T, b_v[None, :], W_skip.T, b_skip[None, :])

    BE = 8000
    e = pl.pallas_call(
        _edge_emb_body,
        grid=(E // BE,),
        in_specs=[
            pl.BlockSpec((BE, 16), lambda i: (i, 0)),
            pl.BlockSpec((16, HID), lambda i: (0, 0)),
            pl.BlockSpec((1, HID), lambda i: (0, 0)),
        ],
        out_specs=pl.BlockSpec((BE, HID), lambda i: (i, 0)),
        out_shape=jax.ShapeDtypeStruct((E, HID), jnp.float32),
    )(edge_attr, W_edge.T, b_edge[None, :])

    part = pl.kernel(
        _sc_edge_pass,
        out_type=jax.ShapeDtypeStruct((NC, NP, MSGW), jnp.float32),
        mesh=_vec_mesh,
        scratch_types=[
            pltpu.VMEM((CHUNK,), jnp.int32),
            pltpu.VMEM((CHUNK,), jnp.int32),
            pltpu.VMEM((CHUNK, HID), jnp.float32),
            pltpu.VMEM((CHUNK, 2 * HID), jnp.float32),
            pltpu.VMEM((CHUNK, HID), jnp.float32),
            pltpu.VMEM((CHUNK, MSGW), jnp.float32),
            pltpu.VMEM_SHARED((NP, MSGW), jnp.float32),
        ],
        compiler_params=_sc_params,
    )(qd, kv, e, src, dst)

    B3 = 1280
    P = pl.pallas_call(
        _combine_body,
        grid=(NP // B3,),
        in_specs=[
            pl.BlockSpec((B3, MSGW), lambda i: (i, 0)),
            pl.BlockSpec((B3, MSGW), lambda i: (i, 0)),
            pl.BlockSpec((B3, HID), lambda i: (i, 0)),
            pl.BlockSpec((HID, NUM_CLASSES), lambda i: (0, 0)),
            pl.BlockSpec((1, NUM_CLASSES), lambda i: (0, 0)),
        ],
        out_specs=pl.BlockSpec((B3, NUM_CLASSES), lambda i: (i, 0)),
        out_shape=jax.ShapeDtypeStruct((NP, NUM_CLASSES), jnp.float32),
    )(part[0], part[1], skip, W_out.T, 0.5 * b_out[None, :])

    edge_preds = pl.kernel(
        _sc_readout,
        out_type=jax.ShapeDtypeStruct((E, NUM_CLASSES), jnp.float32),
        mesh=_vec_mesh,
        scratch_types=[
            pltpu.VMEM((CHUNK,), jnp.int32),
            pltpu.VMEM((CHUNK,), jnp.int32),
            pltpu.VMEM((CHUNK, NUM_CLASSES), jnp.float32),
            pltpu.VMEM((CHUNK, NUM_CLASSES), jnp.float32),
        ],
        compiler_params=_sc_params,
    )(P, src, dst)

    return edge_preds


# speedup vs baseline: 1.3353x; 1.3353x over previous
import dataclasses
import functools

import jax
import jax.numpy as jnp
from jax import lax
from jax.experimental import pallas as pl
from jax.experimental.pallas import tpu as pltpu
from jax.experimental.pallas import tpu_sc as plsc

N = 10000
NP = 10240
E = 320000
HID = 128
HEADS = 8
HD = 16
NUM_CLASSES = 16
MSGW = 144

NC = 2
NS = 16
NW = NC * NS
EPW = E // NW
CHUNK = 40
SUPER = 1000
NSUPER = EPW // SUPER
ROWS_PER_TILE = NP // NS
RCHUNK = 40
NRCHUNK = ROWS_PER_TILE // RCHUNK
CHUNK_R = 80
SUPER_R = 2000

_vec_mesh = plsc.VectorSubcoreMesh(core_axis_name="c", subcore_axis_name="s")

_sc_params = pltpu.CompilerParams(needs_layout_passes=False,
                                  use_tc_tiling_on_sc=False)

_GATHER_DNUMS = lax.GatherDimensionNumbers(
    offset_dims=(), collapsed_slice_dims=(0,), start_index_map=(0,))


def _lane_gather(vec, idx):
    return lax.gather(vec, idx[:, None], _GATHER_DNUMS, slice_sizes=(1,),
                      mode=lax.GatherScatterMode.PROMISE_IN_BOUNDS)


def _node_body(x_ref, wn_ref, bn_ref, wq_ref, bq_ref, wk_ref, bk_ref,
               wv_ref, bv_ref, ws_ref, bs_ref, qd_ref, kv_ref, skip_ref):
    h = jnp.dot(x_ref[...], wn_ref[...], preferred_element_type=jnp.float32) + bn_ref[...]
    q = jnp.dot(h, wq_ref[...], preferred_element_type=jnp.float32) + bq_ref[...]
    qd_ref[...] = q * 0.25
    kv_ref[:, :HID] = jnp.dot(h, wk_ref[...], preferred_element_type=jnp.float32) + bk_ref[...]
    kv_ref[:, HID:] = jnp.dot(h, wv_ref[...], preferred_element_type=jnp.float32) + bv_ref[...]
    skip_ref[...] = jnp.dot(h, ws_ref[...], preferred_element_type=jnp.float32) + bs_ref[...]


def _edge_emb_body(ea_ref, we_ref, be_ref, e_ref):
    e_ref[...] = jnp.dot(ea_ref[...], we_ref[...], preferred_element_type=jnp.float32) + be_ref[...]


def _sc_edge_pass(qd_hbm, kv_hbm, e_hbm, src_hbm, dst_hbm, part_hbm,
                  srcbig, dstbig, dstbuf, qbuf, kvbuf, ebuf, msgbuf, acc, sem):
    cid = lax.axis_index("c")
    sid = lax.axis_index("s")
    wid = sid * NC + cid
    wbase = wid * EPW
    row0 = sid * ROWS_PER_TILE

    lane = lax.iota(jnp.int32, 16)
    idx15 = jnp.full((16,), 15, jnp.int32)

    @pl.loop(0, RCHUNK)
    def _zrow(i):
        for c in range(MSGW // 16):
            msgbuf[i, pl.ds(c * 16, 16)] = jnp.zeros((16,), jnp.float32)

    @pl.loop(0, NRCHUNK)
    def _zcopy(r):
        pltpu.sync_copy(msgbuf, acc.at[pl.ds(row0 + r * RCHUNK, RCHUNK)])

    plsc.subcore_barrier()

    @pl.loop(0, NSUPER)
    def _super(s):
        sbase = pl.multiple_of(wbase + s * SUPER, 8)
        pltpu.sync_copy(src_hbm.at[pl.ds(sbase, SUPER)], srcbig)
        pltpu.sync_copy(dst_hbm.at[pl.ds(sbase, SUPER)], dstbig)

        @pl.loop(0, SUPER // CHUNK)
        def _chunk(c):
            off = pl.multiple_of(c * CHUNK, 8)
            base = pl.multiple_of(sbase + off, 8)
            c1 = pltpu.async_copy(qd_hbm.at[dstbig.at[pl.ds(off, CHUNK)]], qbuf, sem)
            c2 = pltpu.async_copy(kv_hbm.at[srcbig.at[pl.ds(off, CHUNK)]], kvbuf, sem)
            c3 = pltpu.async_copy(e_hbm.at[pl.ds(base, CHUNK)], ebuf, sem)
            c4 = pltpu.async_copy(dst_hbm.at[pl.ds(base, CHUNK)], dstbuf, sem)
            c1.wait()
            c2.wait()
            c3.wait()
            c4.wait()

            @pl.loop(0, CHUNK, step=2)
            def _edge(i):
                for u in range(2):
                    ii = i + u
                    exrow = jnp.zeros((16,), jnp.float32)
                    for h in range(HEADS):
                        sl = pl.ds(h * HD, HD)
                        qv = qbuf[ii, sl]
                        kvv = kvbuf[ii, sl]
                        ev = ebuf[ii, sl]
                        vv = kvbuf[ii, pl.ds(HID + h * HD, HD)]
                        csum = jnp.cumsum(qv * (kvv + ev))
                        sv = _lane_gather(csum, idx15)
                        exv = jnp.exp(sv)
                        msgbuf[ii, sl] = exv * (vv + ev)
                        exrow = jnp.where(lane == h, exv, exrow)
                    msgbuf[ii, pl.ds(HID, 16)] = exrow

            pltpu.sync_copy(msgbuf, acc.at[dstbuf], add=True)

    plsc.subcore_barrier()

    @pl.loop(0, NRCHUNK)
    def _dump(r):
        rr = pl.multiple_of(row0 + r * RCHUNK, 8)
        pltpu.sync_copy(acc.at[pl.ds(rr, RCHUNK)], msgbuf)
        pltpu.sync_copy(msgbuf, part_hbm.at[cid, pl.ds(rr, RCHUNK)])


def _combine_body(p0_ref, p1_ref, skip_ref, wo_ref, bo_ref, p_ref):
    agg = p0_ref[:, :HID] + p1_ref[:, :HID]
    den = p0_ref[:, HID:HID + 16] + p1_ref[:, HID:HID + 16]
    j_iota = lax.broadcasted_iota(jnp.int32, (16, HID), 0)
    c_iota = lax.broadcasted_iota(jnp.int32, (16, HID), 1)
    expand = (c_iota // HD == j_iota).astype(jnp.float32)
    den_exp = jnp.dot(den, expand, preferred_element_type=jnp.float32)
    h2 = agg / (den_exp + 1e-16) + skip_ref[...]
    p_ref[...] = jnp.dot(h2, wo_ref[...], preferred_element_type=jnp.float32) + bo_ref[...]


def _sc_readout(p_hbm, src_hbm, dst_hbm, out_hbm, srcbig, dstbig, abuf, bbuf, sem):
    cid = lax.axis_index("c")
    sid = lax.axis_index("s")
    wid = sid * NC + cid
    wbase = wid * EPW

    @pl.loop(0, EPW // SUPER_R)
    def _super(s):
        sbase = pl.multiple_of(wbase + s * SUPER_R, 8)
        pltpu.sync_copy(src_hbm.at[pl.ds(sbase, SUPER_R)], srcbig)
        pltpu.sync_copy(dst_hbm.at[pl.ds(sbase, SUPER_R)], dstbig)

        @pl.loop(0, SUPER_R // CHUNK_R)
        def _chunk(c):
            off = pl.multiple_of(c * CHUNK_R, 8)
            c1 = pltpu.async_copy(p_hbm.at[srcbig.at[pl.ds(off, CHUNK_R)]], abuf, sem)
            c2 = pltpu.async_copy(p_hbm.at[dstbig.at[pl.ds(off, CHUNK_R)]], bbuf, sem)
            c1.wait()
            c2.wait()

            @pl.loop(0, CHUNK_R, step=4)
            def _edge(i):
                for u in range(4):
                    abuf[i + u, :] = abuf[i + u, :] + bbuf[i + u, :]

            pltpu.sync_copy(abuf, out_hbm.at[pl.ds(sbase + off, CHUNK_R)])


def kernel(x, edge_index, edge_attr, W_node, b_node, W_edge, b_edge, W_q, b_q,
           W_k, b_k, W_v, b_v, W_skip, b_skip, W_out, b_out):
    src = edge_index[0]
    dst = edge_index[1]
    xp = jnp.pad(x, ((0, NP - N), (0, 0)))

    B1 = 1280
    qd, kv, skip = pl.pallas_call(
        _node_body,
        grid=(NP // B1,),
        in_specs=[pl.BlockSpec((B1, HID), lambda i: (i, 0))]
        + [pl.BlockSpec((HID, HID), lambda i: (0, 0)), pl.BlockSpec((1, HID), lambda i: (0, 0))] * 5,
        out_specs=[
            pl.BlockSpec((B1, HID), lambda i: (i, 0)),
            pl.BlockSpec((B1, 2 * HID), lambda i: (i, 0)),
            pl.BlockSpec((B1, HID), lambda i: (i, 0)),
        ],
        out_shape=[
            jax.ShapeDtypeStruct((NP, HID), jnp.float32),
            jax.ShapeDtypeStruct((NP, 2 * HID), jnp.float32),
            jax.ShapeDtypeStruct((NP, HID), jnp.float32),
        ],
    )(xp, W_node.T, b_node[None, :], W_q.T, b_q[None, :], W_k.T, b_k[None, :],
      W_v.T, b_v[None, :], W_skip.T, b_skip[None, :])

    BE = 8000
    e = pl.pallas_call(
        _edge_emb_body,
        grid=(E // BE,),
        in_specs=[
            pl.BlockSpec((BE, 16), lambda i: (i, 0)),
            pl.BlockSpec((16, HID), lambda i: (0, 0)),
            pl.BlockSpec((1, HID), lambda i: (0, 0)),
        ],
        out_specs=pl.BlockSpec((BE, HID), lambda i: (i, 0)),
        out_shape=jax.ShapeDtypeStruct((E, HID), jnp.float32),
    )(edge_attr, W_edge.T, b_edge[None, :])

    part = pl.kernel(
        _sc_edge_pass,
        out_type=jax.ShapeDtypeStruct((NC, NP, MSGW), jnp.float32),
        mesh=_vec_mesh,
        scratch_types=[
            pltpu.VMEM((SUPER,), jnp.int32),
            pltpu.VMEM((SUPER,), jnp.int32),
            pltpu.VMEM((CHUNK,), jnp.int32),
            pltpu.VMEM((CHUNK, HID), jnp.float32),
            pltpu.VMEM((CHUNK, 2 * HID), jnp.float32),
            pltpu.VMEM((CHUNK, HID), jnp.float32),
            pltpu.VMEM((CHUNK, MSGW), jnp.float32),
            pltpu.VMEM_SHARED((NP, MSGW), jnp.float32),
            pltpu.SemaphoreType.DMA,
        ],
        compiler_params=_sc_params,
    )(qd, kv, e, src, dst)

    B3 = 1280
    P = pl.pallas_call(
        _combine_body,
        grid=(NP // B3,),
        in_specs=[
            pl.BlockSpec((B3, MSGW), lambda i: (i, 0)),
            pl.BlockSpec((B3, MSGW), lambda i: (i, 0)),
            pl.BlockSpec((B3, HID), lambda i: (i, 0)),
            pl.BlockSpec((HID, NUM_CLASSES), lambda i: (0, 0)),
            pl.BlockSpec((1, NUM_CLASSES), lambda i: (0, 0)),
        ],
        out_specs=pl.BlockSpec((B3, NUM_CLASSES), lambda i: (i, 0)),
        out_shape=jax.ShapeDtypeStruct((NP, NUM_CLASSES), jnp.float32),
    )(part[0], part[1], skip, W_out.T, 0.5 * b_out[None, :])

    edge_preds = pl.kernel(
        _sc_readout,
        out_type=jax.ShapeDtypeStruct((E, NUM_CLASSES), jnp.float32),
        mesh=_vec_mesh,
        scratch_types=[
            pltpu.VMEM((SUPER_R,), jnp.int32),
            pltpu.VMEM((SUPER_R,), jnp.int32),
            pltpu.VMEM((CHUNK_R, NUM_CLASSES), jnp.float32),
            pltpu.VMEM((CHUNK_R, NUM_CLASSES), jnp.float32),
            pltpu.SemaphoreType.DMA,
        ],
        compiler_params=_sc_params,
    )(P, src, dst)

    return edge_preds


# speedup vs baseline: 2.5911x; 1.9405x over previous
import dataclasses
import functools

import jax
import jax.numpy as jnp
from jax import lax
from jax.experimental import pallas as pl
from jax.experimental.pallas import tpu as pltpu
from jax.experimental.pallas import tpu_sc as plsc

N = 10000
NP = 10240
E = 320000
HID = 128
HEADS = 8
HD = 16
NUM_CLASSES = 16
MSGW = 144

NC = 2
NS = 16
NW = NC * NS
EPW = E // NW
CHUNK = 40
SUPER = 1000
NSUPER = EPW // SUPER
ROWS_PER_TILE = NP // NS
RCHUNK = 40
NRCHUNK = ROWS_PER_TILE // RCHUNK
CHUNK_R = 80
SUPER_R = 2000

_vec_mesh = plsc.VectorSubcoreMesh(core_axis_name="c", subcore_axis_name="s")

_sc_params = pltpu.CompilerParams(needs_layout_passes=False,
                                  use_tc_tiling_on_sc=False)

_GATHER_DNUMS = lax.GatherDimensionNumbers(
    offset_dims=(), collapsed_slice_dims=(0,), start_index_map=(0,))


def _lane_gather(vec, idx):
    return lax.gather(vec, idx[:, None], _GATHER_DNUMS, slice_sizes=(1,),
                      mode=lax.GatherScatterMode.PROMISE_IN_BOUNDS)


def _node_body(x_ref, wn_ref, bn_ref, wq_ref, bq_ref, wk_ref, bk_ref,
               wv_ref, bv_ref, ws_ref, bs_ref, qd_ref, kv_ref, skip_ref):
    h = jnp.dot(x_ref[...], wn_ref[...], preferred_element_type=jnp.float32) + bn_ref[...]
    q = jnp.dot(h, wq_ref[...], preferred_element_type=jnp.float32) + bq_ref[...]
    qd_ref[...] = q * 0.25
    kv_ref[:, :HID] = jnp.dot(h, wk_ref[...], preferred_element_type=jnp.float32) + bk_ref[...]
    kv_ref[:, HID:] = jnp.dot(h, wv_ref[...], preferred_element_type=jnp.float32) + bv_ref[...]
    skip_ref[...] = jnp.dot(h, ws_ref[...], preferred_element_type=jnp.float32) + bs_ref[...]


def _edge_emb_body(ea_ref, we_ref, be_ref, e_ref):
    e_ref[...] = jnp.dot(ea_ref[...], we_ref[...], preferred_element_type=jnp.float32) + be_ref[...]


def _sc_edge_pass(qd_hbm, kv_hbm, e_hbm, ei_hbm, part_hbm,
                  srcbig, dstbig, dstbuf, qbuf, kvbuf, ebuf, msgbuf, acc, sem):
    cid = lax.axis_index("c")
    sid = lax.axis_index("s")
    wid = sid * NC + cid
    wbase = wid * EPW
    row0 = sid * ROWS_PER_TILE

    lane = lax.iota(jnp.int32, 16)
    idx15 = jnp.full((16,), 15, jnp.int32)

    @pl.loop(0, RCHUNK)
    def _zrow(i):
        for c in range(MSGW // 16):
            msgbuf[i, pl.ds(c * 16, 16)] = jnp.zeros((16,), jnp.float32)

    @pl.loop(0, NRCHUNK)
    def _zcopy(r):
        pltpu.sync_copy(msgbuf, acc.at[pl.ds(row0 + r * RCHUNK, RCHUNK)])

    plsc.subcore_barrier()

    @pl.loop(0, NSUPER)
    def _super(s):
        sbase = pl.multiple_of(wbase + s * SUPER, 8)
        pltpu.sync_copy(ei_hbm.at[0, pl.ds(sbase, SUPER)], srcbig)
        pltpu.sync_copy(ei_hbm.at[1, pl.ds(sbase, SUPER)], dstbig)

        @pl.loop(0, SUPER // CHUNK)
        def _chunk(c):
            off = pl.multiple_of(c * CHUNK, 8)
            base = pl.multiple_of(sbase + off, 8)
            c1 = pltpu.async_copy(qd_hbm.at[dstbig.at[pl.ds(off, CHUNK)]], qbuf, sem)
            c2 = pltpu.async_copy(kv_hbm.at[srcbig.at[pl.ds(off, CHUNK)]], kvbuf, sem)
            c3 = pltpu.async_copy(e_hbm.at[pl.ds(base, CHUNK)], ebuf, sem)
            c4 = pltpu.async_copy(ei_hbm.at[1, pl.ds(base, CHUNK)], dstbuf, sem)
            c1.wait()
            c2.wait()
            c3.wait()
            c4.wait()

            @pl.loop(0, CHUNK, step=2)
            def _edge(i):
                for u in range(2):
                    ii = i + u
                    ke = []
                    ve = []
                    for h in range(HEADS):
                        sl = pl.ds(h * HD, HD)
                        kvv = kvbuf[ii, sl]
                        ev = ebuf[ii, sl]
                        vv = kvbuf[ii, pl.ds(HID + h * HD, HD)]
                        ke.append(qbuf[ii, sl] * (kvv + ev))
                        ve.append(vv + ev)
                    cs = [jnp.cumsum(t) for t in ke]
                    sv = [_lane_gather(csx, idx15) for csx in cs]
                    ex = [jnp.exp(s) for s in sv]
                    w = [jnp.where(lane == h, ex[h], 0.0) for h in range(HEADS)]
                    for h in range(HEADS):
                        msgbuf[ii, pl.ds(h * HD, HD)] = ex[h] * ve[h]
                    w4 = [w[0] + w[1], w[2] + w[3], w[4] + w[5], w[6] + w[7]]
                    msgbuf[ii, pl.ds(HID, 16)] = (w4[0] + w4[1]) + (w4[2] + w4[3])

            pltpu.sync_copy(msgbuf, acc.at[dstbuf], add=True)

    plsc.subcore_barrier()

    @pl.loop(0, NRCHUNK)
    def _dump(r):
        rr = pl.multiple_of(row0 + r * RCHUNK, 8)
        pltpu.sync_copy(acc.at[pl.ds(rr, RCHUNK)], msgbuf)
        pltpu.sync_copy(msgbuf, part_hbm.at[cid, pl.ds(rr, RCHUNK)])


def _combine_body(p0_ref, p1_ref, skip_ref, wo_ref, bo_ref, p_ref):
    agg = p0_ref[0, :, :HID] + p1_ref[0, :, :HID]
    den = p0_ref[0, :, HID:HID + 16] + p1_ref[0, :, HID:HID + 16]
    j_iota = lax.broadcasted_iota(jnp.int32, (16, HID), 0)
    c_iota = lax.broadcasted_iota(jnp.int32, (16, HID), 1)
    expand = (c_iota // HD == j_iota).astype(jnp.float32)
    den_exp = jnp.dot(den, expand, preferred_element_type=jnp.float32)
    h2 = agg / (den_exp + 1e-16) + skip_ref[...]
    p_ref[...] = jnp.dot(h2, wo_ref[...], preferred_element_type=jnp.float32) + bo_ref[...]


def _sc_readout(p_hbm, ei_hbm, out_hbm, srcbig, dstbig, abuf, bbuf, obuf, sem):
    cid = lax.axis_index("c")
    sid = lax.axis_index("s")
    wid = sid * NC + cid
    wbase = wid * EPW

    @pl.loop(0, EPW // SUPER_R)
    def _super(s):
        sbase = pl.multiple_of(wbase + s * SUPER_R, 8)
        pltpu.sync_copy(ei_hbm.at[0, pl.ds(sbase, SUPER_R)], srcbig)
        pltpu.sync_copy(ei_hbm.at[1, pl.ds(sbase, SUPER_R)], dstbig)

        @pl.loop(0, SUPER_R // CHUNK_R)
        def _chunk(c):
            off = pl.multiple_of(c * CHUNK_R, 8)
            c1 = pltpu.async_copy(p_hbm.at[srcbig.at[pl.ds(off, CHUNK_R)]], abuf, sem)
            c2 = pltpu.async_copy(p_hbm.at[dstbig.at[pl.ds(off, CHUNK_R)]], bbuf, sem)
            c1.wait()
            c2.wait()

            @pl.loop(0, CHUNK_R, step=4)
            def _edge(i):
                for u in range(4):
                    ii = i + u
                    obuf[lax.shift_right_logical(ii, 3),
                         pl.ds(lax.shift_left(jnp.bitwise_and(ii, 7), 4), 16)] = (
                        abuf[ii, :] + bbuf[ii, :])

            pltpu.sync_copy(obuf, out_hbm.at[pl.ds((sbase + off) // 8, CHUNK_R // 8)])


def kernel(x, edge_index, edge_attr, W_node, b_node, W_edge, b_edge, W_q, b_q,
           W_k, b_k, W_v, b_v, W_skip, b_skip, W_out, b_out):
    B1 = 1000
    qd, kv, skip = pl.pallas_call(
        _node_body,
        grid=(N // B1,),
        in_specs=[pl.BlockSpec((B1, HID), lambda i: (i, 0))]
        + [pl.BlockSpec((HID, HID), lambda i: (0, 0)), pl.BlockSpec((1, HID), lambda i: (0, 0))] * 5,
        out_specs=[
            pl.BlockSpec((B1, HID), lambda i: (i, 0)),
            pl.BlockSpec((B1, 2 * HID), lambda i: (i, 0)),
            pl.BlockSpec((B1, HID), lambda i: (i, 0)),
        ],
        out_shape=[
            jax.ShapeDtypeStruct((N, HID), jnp.float32),
            jax.ShapeDtypeStruct((N, 2 * HID), jnp.float32),
            jax.ShapeDtypeStruct((N, HID), jnp.float32),
        ],
    )(x, W_node.T, b_node[None, :], W_q.T, b_q[None, :], W_k.T, b_k[None, :],
      W_v.T, b_v[None, :], W_skip.T, b_skip[None, :])

    BE = 8000
    e = pl.pallas_call(
        _edge_emb_body,
        grid=(E // BE,),
        in_specs=[
            pl.BlockSpec((BE, 16), lambda i: (i, 0)),
            pl.BlockSpec((16, HID), lambda i: (0, 0)),
            pl.BlockSpec((1, HID), lambda i: (0, 0)),
        ],
        out_specs=pl.BlockSpec((BE, HID), lambda i: (i, 0)),
        out_shape=jax.ShapeDtypeStruct((E, HID), jnp.float32),
    )(edge_attr, W_edge.T, b_edge[None, :])

    part = pl.kernel(
        _sc_edge_pass,
        out_type=jax.ShapeDtypeStruct((NC, NP, MSGW), jnp.float32),
        mesh=_vec_mesh,
        scratch_types=[
            pltpu.VMEM((SUPER,), jnp.int32),
            pltpu.VMEM((SUPER,), jnp.int32),
            pltpu.VMEM((CHUNK,), jnp.int32),
            pltpu.VMEM((CHUNK, HID), jnp.float32),
            pltpu.VMEM((CHUNK, 2 * HID), jnp.float32),
            pltpu.VMEM((CHUNK, HID), jnp.float32),
            pltpu.VMEM((CHUNK, MSGW), jnp.float32),
            pltpu.VMEM_SHARED((NP, MSGW), jnp.float32),
            pltpu.SemaphoreType.DMA,
        ],
        compiler_params=_sc_params,
    )(qd, kv, e, edge_index)

    B3 = 1000
    P = pl.pallas_call(
        _combine_body,
        grid=(N // B3,),
        in_specs=[
            pl.BlockSpec((1, B3, MSGW), lambda i: (0, i, 0)),
            pl.BlockSpec((1, B3, MSGW), lambda i: (1, i, 0)),
            pl.BlockSpec((B3, HID), lambda i: (i, 0)),
            pl.BlockSpec((HID, NUM_CLASSES), lambda i: (0, 0)),
            pl.BlockSpec((1, NUM_CLASSES), lambda i: (0, 0)),
        ],
        out_specs=pl.BlockSpec((B3, NUM_CLASSES), lambda i: (i, 0)),
        out_shape=jax.ShapeDtypeStruct((N, NUM_CLASSES), jnp.float32),
    )(part, part, skip, W_out.T, 0.5 * b_out[None, :])

    packed = pl.kernel(
        _sc_readout,
        out_type=jax.ShapeDtypeStruct((E // 8, 8 * NUM_CLASSES), jnp.float32),
        mesh=_vec_mesh,
        scratch_types=[
            pltpu.VMEM((SUPER_R,), jnp.int32),
            pltpu.VMEM((SUPER_R,), jnp.int32),
            pltpu.VMEM((CHUNK_R, NUM_CLASSES), jnp.float32),
            pltpu.VMEM((CHUNK_R, NUM_CLASSES), jnp.float32),
            pltpu.VMEM((CHUNK_R // 8, 8 * NUM_CLASSES), jnp.float32),
            pltpu.SemaphoreType.DMA,
        ],
        compiler_params=_sc_params,
    )(P, edge_index)

    return packed.reshape(E, NUM_CLASSES)


# speedup vs baseline: 2.6650x; 1.0285x over previous
import dataclasses
import functools

import jax
import jax.numpy as jnp
from jax import lax
from jax.experimental import pallas as pl
from jax.experimental.pallas import tpu as pltpu
from jax.experimental.pallas import tpu_sc as plsc

N = 10000
NP = 10240
E = 320000
HID = 128
HEADS = 8
HD = 16
NUM_CLASSES = 16
MSGW = 144

NC = 2
NS = 16
NW = NC * NS
EPW = E // NW
CHUNK = 40
SUPER = 1000
NSUPER = EPW // SUPER
ROWS_PER_TILE = NP // NS
RCHUNK = 40
NRCHUNK = ROWS_PER_TILE // RCHUNK
CHUNK_R = 80
SUPER_R = 2000

_vec_mesh = plsc.VectorSubcoreMesh(core_axis_name="c", subcore_axis_name="s")

_sc_params = pltpu.CompilerParams(needs_layout_passes=False,
                                  use_tc_tiling_on_sc=False)

_GATHER_DNUMS = lax.GatherDimensionNumbers(
    offset_dims=(), collapsed_slice_dims=(0,), start_index_map=(0,))


def _lane_gather(vec, idx):
    return lax.gather(vec, idx[:, None], _GATHER_DNUMS, slice_sizes=(1,),
                      mode=lax.GatherScatterMode.PROMISE_IN_BOUNDS)


def _node_body(x_ref, wn_ref, bn_ref, wq_ref, bq_ref, wk_ref, bk_ref,
               wv_ref, bv_ref, ws_ref, bs_ref, qd_ref, kv_ref, skip_ref):
    h = jnp.dot(x_ref[...], wn_ref[...], preferred_element_type=jnp.float32) + bn_ref[...]
    q = jnp.dot(h, wq_ref[...], preferred_element_type=jnp.float32) + bq_ref[...]
    qd_ref[...] = q * 0.25
    kv_ref[:, :HID] = jnp.dot(h, wk_ref[...], preferred_element_type=jnp.float32) + bk_ref[...]
    kv_ref[:, HID:] = jnp.dot(h, wv_ref[...], preferred_element_type=jnp.float32) + bv_ref[...]
    skip_ref[...] = jnp.dot(h, ws_ref[...], preferred_element_type=jnp.float32) + bs_ref[...]


def _edge_emb_body(ea_ref, we_ref, be_ref, e_ref):
    e_ref[...] = jnp.dot(ea_ref[...], we_ref[...], preferred_element_type=jnp.float32) + be_ref[...]


def _sc_edge_pass(qd_hbm, kv_hbm, e_hbm, ei_hbm, part_hbm,
                  srcbig, dstbig, dstbuf, qbuf, kvbuf, ebuf, msgbuf, acc, sem):
    cid = lax.axis_index("c")
    sid = lax.axis_index("s")
    wid = sid * NC + cid
    wbase = wid * EPW
    row0 = sid * ROWS_PER_TILE

    lane = lax.iota(jnp.int32, 16)
    idx15 = jnp.full((16,), 15, jnp.int32)

    @pl.loop(0, RCHUNK)
    def _zrow(i):
        for c in range(MSGW // 16):
            msgbuf[i, pl.ds(c * 16, 16)] = jnp.zeros((16,), jnp.float32)

    @pl.loop(0, NRCHUNK)
    def _zcopy(r):
        pltpu.sync_copy(msgbuf, acc.at[pl.ds(row0 + r * RCHUNK, RCHUNK)])

    plsc.subcore_barrier()

    @pl.loop(0, NSUPER)
    def _super(s):
        sbase = pl.multiple_of(wbase + s * SUPER, 8)
        pltpu.sync_copy(ei_hbm.at[0, pl.ds(sbase, SUPER)], srcbig)
        pltpu.sync_copy(ei_hbm.at[1, pl.ds(sbase, SUPER)], dstbig)

        @pl.loop(0, SUPER // CHUNK)
        def _chunk(c):
            off = pl.multiple_of(c * CHUNK, 8)
            base = pl.multiple_of(sbase + off, 8)
            c1 = pltpu.async_copy(qd_hbm.at[dstbig.at[pl.ds(off, CHUNK)]], qbuf, sem)
            c2 = pltpu.async_copy(kv_hbm.at[srcbig.at[pl.ds(off, CHUNK)]], kvbuf, sem)
            c3 = pltpu.async_copy(e_hbm.at[pl.ds(base * HID, CHUNK * HID)], ebuf, sem)
            c4 = pltpu.async_copy(ei_hbm.at[1, pl.ds(base, CHUNK)], dstbuf, sem)
            c1.wait()
            c2.wait()
            c3.wait()
            c4.wait()

            @pl.loop(0, CHUNK, step=2)
            def _edge(i):
                for u in range(2):
                    ii = i + u
                    ke = []
                    ve = []
                    for h in range(HEADS):
                        sl = pl.ds(h * HD, HD)
                        kvv = kvbuf[ii, sl]
                        ev = ebuf[pl.ds(ii * HID + h * HD, HD)]
                        vv = kvbuf[ii, pl.ds(HID + h * HD, HD)]
                        ke.append(qbuf[ii, sl] * (kvv + ev))
                        ve.append(vv + ev)
                    cs = [jnp.cumsum(t) for t in ke]
                    sv = [_lane_gather(csx, idx15) for csx in cs]
                    ex = [jnp.exp(s) for s in sv]
                    w = [jnp.where(lane == h, ex[h], 0.0) for h in range(HEADS)]
                    for h in range(HEADS):
                        msgbuf[ii, pl.ds(h * HD, HD)] = ex[h] * ve[h]
                    w4 = [w[0] + w[1], w[2] + w[3], w[4] + w[5], w[6] + w[7]]
                    msgbuf[ii, pl.ds(HID, 16)] = (w4[0] + w4[1]) + (w4[2] + w4[3])

            pltpu.sync_copy(msgbuf, acc.at[dstbuf], add=True)

    plsc.subcore_barrier()

    @pl.loop(0, NRCHUNK)
    def _dump(r):
        rr = pl.multiple_of(row0 + r * RCHUNK, 8)
        pltpu.sync_copy(acc.at[pl.ds(rr, RCHUNK)], msgbuf)
        pltpu.sync_copy(msgbuf, part_hbm.at[cid, pl.ds(rr, RCHUNK)])


def _combine_body(p0_ref, p1_ref, skip_ref, wo_ref, bo_ref, p_ref):
    agg = p0_ref[0, :, :HID] + p1_ref[0, :, :HID]
    den = p0_ref[0, :, HID:HID + 16] + p1_ref[0, :, HID:HID + 16]
    j_iota = lax.broadcasted_iota(jnp.int32, (16, HID), 0)
    c_iota = lax.broadcasted_iota(jnp.int32, (16, HID), 1)
    expand = (c_iota // HD == j_iota).astype(jnp.float32)
    den_exp = jnp.dot(den, expand, preferred_element_type=jnp.float32)
    h2 = agg / (den_exp + 1e-16) + skip_ref[...]
    p_ref[...] = jnp.dot(h2, wo_ref[...], preferred_element_type=jnp.float32) + bo_ref[...]


def _sc_readout(p_hbm, ei_hbm, out_hbm, srcbig, dstbig, abuf, bbuf, sem):
    cid = lax.axis_index("c")
    sid = lax.axis_index("s")
    wid = sid * NC + cid
    wbase = wid * EPW

    @pl.loop(0, EPW // SUPER_R)
    def _super(s):
        sbase = pl.multiple_of(wbase + s * SUPER_R, 8)
        pltpu.sync_copy(ei_hbm.at[0, pl.ds(sbase, SUPER_R)], srcbig)
        pltpu.sync_copy(ei_hbm.at[1, pl.ds(sbase, SUPER_R)], dstbig)

        @pl.loop(0, SUPER_R // CHUNK_R)
        def _chunk(c):
            off = pl.multiple_of(c * CHUNK_R, 8)
            c1 = pltpu.async_copy(p_hbm.at[srcbig.at[pl.ds(off, CHUNK_R)]], abuf, sem)
            c2 = pltpu.async_copy(p_hbm.at[dstbig.at[pl.ds(off, CHUNK_R)]], bbuf, sem)
            c1.wait()
            c2.wait()

            @pl.loop(0, CHUNK_R, step=4)
            def _edge(i):
                for u in range(4):
                    abuf[i + u, :] = abuf[i + u, :] + bbuf[i + u, :]

            pltpu.sync_copy(abuf, out_hbm.at[pl.ds(sbase + off, CHUNK_R)])


def kernel(x, edge_index, edge_attr, W_node, b_node, W_edge, b_edge, W_q, b_q,
           W_k, b_k, W_v, b_v, W_skip, b_skip, W_out, b_out):
    B1 = 1000
    qd, kv, skip = pl.pallas_call(
        _node_body,
        grid=(N // B1,),
        in_specs=[pl.BlockSpec((B1, HID), lambda i: (i, 0))]
        + [pl.BlockSpec((HID, HID), lambda i: (0, 0)), pl.BlockSpec((1, HID), lambda i: (0, 0))] * 5,
        out_specs=[
            pl.BlockSpec((B1, HID), lambda i: (i, 0)),
            pl.BlockSpec((B1, 2 * HID), lambda i: (i, 0)),
            pl.BlockSpec((B1, HID), lambda i: (i, 0)),
        ],
        out_shape=[
            jax.ShapeDtypeStruct((N, HID), jnp.float32),
            jax.ShapeDtypeStruct((N, 2 * HID), jnp.float32),
            jax.ShapeDtypeStruct((N, HID), jnp.float32),
        ],
    )(x, W_node.T, b_node[None, :], W_q.T, b_q[None, :], W_k.T, b_k[None, :],
      W_v.T, b_v[None, :], W_skip.T, b_skip[None, :])

    BE = 8000
    e = pl.pallas_call(
        _edge_emb_body,
        grid=(E // BE,),
        in_specs=[
            pl.BlockSpec((BE, 16), lambda i: (i, 0)),
            pl.BlockSpec((16, HID), lambda i: (0, 0)),
            pl.BlockSpec((1, HID), lambda i: (0, 0)),
        ],
        out_specs=pl.BlockSpec((BE, HID), lambda i: (i, 0)),
        out_shape=jax.ShapeDtypeStruct((E, HID), jnp.float32),
    )(edge_attr, W_edge.T, b_edge[None, :])

    part = pl.kernel(
        _sc_edge_pass,
        out_type=jax.ShapeDtypeStruct((NC, NP, MSGW), jnp.float32),
        mesh=_vec_mesh,
        scratch_types=[
            pltpu.VMEM((SUPER,), jnp.int32),
            pltpu.VMEM((SUPER,), jnp.int32),
            pltpu.VMEM((CHUNK,), jnp.int32),
            pltpu.VMEM((CHUNK, HID), jnp.float32),
            pltpu.VMEM((CHUNK, 2 * HID), jnp.float32),
            pltpu.VMEM((CHUNK * HID,), jnp.float32),
            pltpu.VMEM((CHUNK, MSGW), jnp.float32),
            pltpu.VMEM_SHARED((NP, MSGW), jnp.float32),
            pltpu.SemaphoreType.DMA,
        ],
        compiler_params=_sc_params,
    )(qd, kv, e.reshape(-1), edge_index)

    B3 = 1000
    P = pl.pallas_call(
        _combine_body,
        grid=(N // B3,),
        in_specs=[
            pl.BlockSpec((1, B3, MSGW), lambda i: (0, i, 0)),
            pl.BlockSpec((1, B3, MSGW), lambda i: (1, i, 0)),
            pl.BlockSpec((B3, HID), lambda i: (i, 0)),
            pl.BlockSpec((HID, NUM_CLASSES), lambda i: (0, 0)),
            pl.BlockSpec((1, NUM_CLASSES), lambda i: (0, 0)),
        ],
        out_specs=pl.BlockSpec((B3, NUM_CLASSES), lambda i: (i, 0)),
        out_shape=jax.ShapeDtypeStruct((N, NUM_CLASSES), jnp.float32),
    )(part, part, skip, W_out.T, 0.5 * b_out[None, :])

    edge_preds = pl.kernel(
        _sc_readout,
        out_type=jax.ShapeDtypeStruct((E, NUM_CLASSES), jnp.float32),
        mesh=_vec_mesh,
        scratch_types=[
            pltpu.VMEM((SUPER_R,), jnp.int32),
            pltpu.VMEM((SUPER_R,), jnp.int32),
            pltpu.VMEM((CHUNK_R, NUM_CLASSES), jnp.float32),
            pltpu.VMEM((CHUNK_R, NUM_CLASSES), jnp.float32),
            pltpu.SemaphoreType.DMA,
        ],
        compiler_params=_sc_params,
    )(P, edge_index)

    return edge_preds


# speedup vs baseline: 2.7739x; 1.0409x over previous
import dataclasses
import functools

import jax
import jax.numpy as jnp
from jax import lax
from jax.experimental import pallas as pl
from jax.experimental.pallas import tpu as pltpu
from jax.experimental.pallas import tpu_sc as plsc

N = 10000
NP = 10240
E = 320000
HID = 128
HEADS = 8
HD = 16
NUM_CLASSES = 16
MSGW = 144

NC = 2
NS = 16
NW = NC * NS
EPW = E // NW
CHUNK = 40
SUPER = 1000
NSUPER = EPW // SUPER
ROWS_PER_TILE = NP // NS
RCHUNK = 40
NRCHUNK = ROWS_PER_TILE // RCHUNK
CHUNK_R = 80
SUPER_R = 2000

_vec_mesh = plsc.VectorSubcoreMesh(core_axis_name="c", subcore_axis_name="s")

_sc_params = pltpu.CompilerParams(needs_layout_passes=False,
                                  use_tc_tiling_on_sc=False)

_GATHER_DNUMS = lax.GatherDimensionNumbers(
    offset_dims=(), collapsed_slice_dims=(0,), start_index_map=(0,))


def _lane_gather(vec, idx):
    return lax.gather(vec, idx[:, None], _GATHER_DNUMS, slice_sizes=(1,),
                      mode=lax.GatherScatterMode.PROMISE_IN_BOUNDS)


def _node_body(x_ref, wn_ref, bn_ref, wq_ref, bq_ref, wk_ref, bk_ref,
               wv_ref, bv_ref, ws_ref, bs_ref, qd_ref, kv_ref, skip_ref):
    h = jnp.dot(x_ref[...], wn_ref[...], preferred_element_type=jnp.float32) + bn_ref[...]
    q = jnp.dot(h, wq_ref[...], preferred_element_type=jnp.float32) + bq_ref[...]
    qd_ref[...] = q * 0.25
    kv_ref[:, :HID] = jnp.dot(h, wk_ref[...], preferred_element_type=jnp.float32) + bk_ref[...]
    kv_ref[:, HID:] = jnp.dot(h, wv_ref[...], preferred_element_type=jnp.float32) + bv_ref[...]
    skip_ref[...] = jnp.dot(h, ws_ref[...], preferred_element_type=jnp.float32) + bs_ref[...]


def _edge_emb_body(ea_ref, we_ref, be_ref, e_ref):
    e_ref[...] = jnp.dot(ea_ref[...], we_ref[...], preferred_element_type=jnp.float32) + be_ref[...]


def _sc_edge_pass(qd_hbm, kv_hbm, e_hbm, ei_hbm, part_hbm,
                  srcA, srcB, dstA, dstB, qbuf, kvA, kvB, ebuf, msgbuf, acc,
                  semIA, semIB, semKA, semKB, semQ):
    cid = lax.axis_index("c")
    sid = lax.axis_index("s")
    wid = sid * NC + cid
    wbase = wid * EPW
    row0 = sid * ROWS_PER_TILE

    lane = lax.iota(jnp.int32, 16)
    idx15 = jnp.full((16,), 15, jnp.int32)

    @pl.loop(0, RCHUNK)
    def _zrow(i):
        for c in range(MSGW // 16):
            msgbuf[i, pl.ds(c * 16, 16)] = jnp.zeros((16,), jnp.float32)

    @pl.loop(0, NRCHUNK)
    def _zcopy(r):
        pltpu.sync_copy(msgbuf, acc.at[pl.ds(row0 + r * RCHUNK, RCHUNK)])

    plsc.subcore_barrier()

    NPAIR = EPW // (2 * CHUNK)

    def _idx_fire(base, sbuf, dbuf, semx):
        pltpu.async_copy(ei_hbm.at[0, pl.ds(base, CHUNK)], sbuf, semx)
        pltpu.async_copy(ei_hbm.at[1, pl.ds(base, CHUNK)], dbuf, semx)

    def _idx_drain(base, sbuf, dbuf, semx):
        pltpu.make_async_copy(ei_hbm.at[0, pl.ds(base, CHUNK)], sbuf, semx).wait()
        pltpu.make_async_copy(ei_hbm.at[1, pl.ds(base, CHUNK)], dbuf, semx).wait()

    def _compute(kvbuf, dbuf):
        @pl.loop(0, CHUNK, step=2)
        def _edge(i):
            for u in range(2):
                ii = i + u
                ke = []
                ve = []
                for h in range(HEADS):
                    sl = pl.ds(h * HD, HD)
                    kvv = kvbuf[ii, sl]
                    ev = ebuf[pl.ds(ii * HID + h * HD, HD)]
                    vv = kvbuf[ii, pl.ds(HID + h * HD, HD)]
                    ke.append(qbuf[ii, sl] * (kvv + ev))
                    ve.append(vv + ev)
                cs = [jnp.cumsum(t) for t in ke]
                sv = [_lane_gather(csx, idx15) for csx in cs]
                ex = [jnp.exp(s) for s in sv]
                w = [jnp.where(lane == h, ex[h], 0.0) for h in range(HEADS)]
                for h in range(HEADS):
                    msgbuf[ii, pl.ds(h * HD, HD)] = ex[h] * ve[h]
                w4 = [w[0] + w[1], w[2] + w[3], w[4] + w[5], w[6] + w[7]]
                msgbuf[ii, pl.ds(HID, 16)] = (w4[0] + w4[1]) + (w4[2] + w4[3])

        pltpu.sync_copy(msgbuf, acc.at[dbuf], add=True)

    _idx_fire(wbase, srcA, dstA, semIA)
    _idx_drain(wbase, srcA, dstA, semIA)
    pltpu.async_copy(kv_hbm.at[srcA], kvA, semKA)
    _idx_fire(wbase + CHUNK, srcB, dstB, semIB)

    @pl.loop(0, NPAIR)
    def _pair(p):
        baseA = pl.multiple_of(wbase + p * 2 * CHUNK, 8)
        baseB = pl.multiple_of(baseA + CHUNK, 8)
        pltpu.async_copy(qd_hbm.at[dstA], qbuf, semQ)
        pltpu.async_copy(e_hbm.at[pl.ds(baseA * HID, CHUNK * HID)], ebuf, semQ)
        _idx_drain(baseB, srcB, dstB, semIB)
        pltpu.async_copy(kv_hbm.at[srcB], kvB, semKB)
        pltpu.make_async_copy(kv_hbm.at[srcA], kvA, semKA).wait()
        pltpu.make_async_copy(qd_hbm.at[dstA], qbuf, semQ).wait()
        pltpu.make_async_copy(e_hbm.at[pl.ds(baseA * HID, CHUNK * HID)], ebuf, semQ).wait()
        _compute(kvA, dstA)

        @pl.when(p < NPAIR - 1)
        def _pfA():
            _idx_fire(baseA + 2 * CHUNK, srcA, dstA, semIA)

        pltpu.async_copy(qd_hbm.at[dstB], qbuf, semQ)
        pltpu.async_copy(e_hbm.at[pl.ds(baseB * HID, CHUNK * HID)], ebuf, semQ)
        pltpu.make_async_copy(kv_hbm.at[srcB], kvB, semKB).wait()
        pltpu.make_async_copy(qd_hbm.at[dstB], qbuf, semQ).wait()
        pltpu.make_async_copy(e_hbm.at[pl.ds(baseB * HID, CHUNK * HID)], ebuf, semQ).wait()
        _compute(kvB, dstB)

        @pl.when(p < NPAIR - 1)
        def _pfB():
            _idx_drain(baseA + 2 * CHUNK, srcA, dstA, semIA)
            pltpu.async_copy(kv_hbm.at[srcA], kvA, semKA)
            _idx_fire(baseB + 2 * CHUNK, srcB, dstB, semIB)

    plsc.subcore_barrier()

    @pl.loop(0, NRCHUNK)
    def _dump(r):
        rr = pl.multiple_of(row0 + r * RCHUNK, 8)
        pltpu.sync_copy(acc.at[pl.ds(rr, RCHUNK)], msgbuf)
        pltpu.sync_copy(msgbuf, part_hbm.at[cid, pl.ds(rr, RCHUNK)])


def _combine_body(p0_ref, p1_ref, skip_ref, wo_ref, bo_ref, p_ref):
    agg = p0_ref[0, :, :HID] + p1_ref[0, :, :HID]
    den = p0_ref[0, :, HID:HID + 16] + p1_ref[0, :, HID:HID + 16]
    j_iota = lax.broadcasted_iota(jnp.int32, (16, HID), 0)
    c_iota = lax.broadcasted_iota(jnp.int32, (16, HID), 1)
    expand = (c_iota // HD == j_iota).astype(jnp.float32)
    den_exp = jnp.dot(den, expand, preferred_element_type=jnp.float32)
    h2 = agg / (den_exp + 1e-16) + skip_ref[...]
    p_ref[...] = jnp.dot(h2, wo_ref[...], preferred_element_type=jnp.float32) + bo_ref[...]


def _sc_readout(p_hbm, ei_hbm, out_hbm, srcbig, dstbig, abuf, bbuf, sem):
    cid = lax.axis_index("c")
    sid = lax.axis_index("s")
    wid = sid * NC + cid
    wbase = wid * EPW

    @pl.loop(0, EPW // SUPER_R)
    def _super(s):
        sbase = pl.multiple_of(wbase + s * SUPER_R, 8)
        pltpu.sync_copy(ei_hbm.at[0, pl.ds(sbase, SUPER_R)], srcbig)
        pltpu.sync_copy(ei_hbm.at[1, pl.ds(sbase, SUPER_R)], dstbig)

        @pl.loop(0, SUPER_R // CHUNK_R)
        def _chunk(c):
            off = pl.multiple_of(c * CHUNK_R, 8)
            c1 = pltpu.async_copy(p_hbm.at[srcbig.at[pl.ds(off, CHUNK_R)]], abuf, sem)
            c2 = pltpu.async_copy(p_hbm.at[dstbig.at[pl.ds(off, CHUNK_R)]], bbuf, sem)
            c1.wait()
            c2.wait()

            @pl.loop(0, CHUNK_R, step=4)
            def _edge(i):
                for u in range(4):
                    abuf[i + u, :] = abuf[i + u, :] + bbuf[i + u, :]

            pltpu.sync_copy(abuf, out_hbm.at[pl.ds(sbase + off, CHUNK_R)])


def kernel(x, edge_index, edge_attr, W_node, b_node, W_edge, b_edge, W_q, b_q,
           W_k, b_k, W_v, b_v, W_skip, b_skip, W_out, b_out):
    B1 = 1000
    qd, kv, skip = pl.pallas_call(
        _node_body,
        grid=(N // B1,),
        in_specs=[pl.BlockSpec((B1, HID), lambda i: (i, 0))]
        + [pl.BlockSpec((HID, HID), lambda i: (0, 0)), pl.BlockSpec((1, HID), lambda i: (0, 0))] * 5,
        out_specs=[
            pl.BlockSpec((B1, HID), lambda i: (i, 0)),
            pl.BlockSpec((B1, 2 * HID), lambda i: (i, 0)),
            pl.BlockSpec((B1, HID), lambda i: (i, 0)),
        ],
        out_shape=[
            jax.ShapeDtypeStruct((N, HID), jnp.float32),
            jax.ShapeDtypeStruct((N, 2 * HID), jnp.float32),
            jax.ShapeDtypeStruct((N, HID), jnp.float32),
        ],
    )(x, W_node.T, b_node[None, :], W_q.T, b_q[None, :], W_k.T, b_k[None, :],
      W_v.T, b_v[None, :], W_skip.T, b_skip[None, :])

    BE = 8000
    e = pl.pallas_call(
        _edge_emb_body,
        grid=(E // BE,),
        in_specs=[
            pl.BlockSpec((BE, 16), lambda i: (i, 0)),
            pl.BlockSpec((16, HID), lambda i: (0, 0)),
            pl.BlockSpec((1, HID), lambda i: (0, 0)),
        ],
        out_specs=pl.BlockSpec((BE, HID), lambda i: (i, 0)),
        out_shape=jax.ShapeDtypeStruct((E, HID), jnp.float32),
    )(edge_attr, W_edge.T, b_edge[None, :])

    part = pl.kernel(
        _sc_edge_pass,
        out_type=jax.ShapeDtypeStruct((NC, NP, MSGW), jnp.float32),
        mesh=_vec_mesh,
        scratch_types=[
            pltpu.VMEM((CHUNK,), jnp.int32),
            pltpu.VMEM((CHUNK,), jnp.int32),
            pltpu.VMEM((CHUNK,), jnp.int32),
            pltpu.VMEM((CHUNK,), jnp.int32),
            pltpu.VMEM((CHUNK, HID), jnp.float32),
            pltpu.VMEM((CHUNK, 2 * HID), jnp.float32),
            pltpu.VMEM((CHUNK, 2 * HID), jnp.float32),
            pltpu.VMEM((CHUNK * HID,), jnp.float32),
            pltpu.VMEM((CHUNK, MSGW), jnp.float32),
            pltpu.VMEM_SHARED((NP, MSGW), jnp.float32),
            pltpu.SemaphoreType.DMA,
            pltpu.SemaphoreType.DMA,
            pltpu.SemaphoreType.DMA,
            pltpu.SemaphoreType.DMA,
            pltpu.SemaphoreType.DMA,
        ],
        compiler_params=_sc_params,
    )(qd, kv, e.reshape(-1), edge_index)

    B3 = 1000
    P = pl.pallas_call(
        _combine_body,
        grid=(N // B3,),
        in_specs=[
            pl.BlockSpec((1, B3, MSGW), lambda i: (0, i, 0)),
            pl.BlockSpec((1, B3, MSGW), lambda i: (1, i, 0)),
            pl.BlockSpec((B3, HID), lambda i: (i, 0)),
            pl.BlockSpec((HID, NUM_CLASSES), lambda i: (0, 0)),
            pl.BlockSpec((1, NUM_CLASSES), lambda i: (0, 0)),
        ],
        out_specs=pl.BlockSpec((B3, NUM_CLASSES), lambda i: (i, 0)),
        out_shape=jax.ShapeDtypeStruct((N, NUM_CLASSES), jnp.float32),
    )(part, part, skip, W_out.T, 0.5 * b_out[None, :])

    edge_preds = pl.kernel(
        _sc_readout,
        out_type=jax.ShapeDtypeStruct((E, NUM_CLASSES), jnp.float32),
        mesh=_vec_mesh,
        scratch_types=[
            pltpu.VMEM((SUPER_R,), jnp.int32),
            pltpu.VMEM((SUPER_R,), jnp.int32),
            pltpu.VMEM((CHUNK_R, NUM_CLASSES), jnp.float32),
            pltpu.VMEM((CHUNK_R, NUM_CLASSES), jnp.float32),
            pltpu.SemaphoreType.DMA,
        ],
        compiler_params=_sc_params,
    )(P, edge_index)

    return edge_preds


# speedup vs baseline: 2.8470x; 1.0263x over previous
import dataclasses
import functools

import jax
import jax.numpy as jnp
from jax import lax
from jax.experimental import pallas as pl
from jax.experimental.pallas import tpu as pltpu
from jax.experimental.pallas import tpu_sc as plsc

N = 10000
NP = 10240
E = 320000
HID = 128
HEADS = 8
HD = 16
NUM_CLASSES = 16
MSGW = 144

NC = 2
NS = 16
NW = NC * NS
EPW = E // NW
CHUNK = 40
SUPER = 1000
NSUPER = EPW // SUPER
ROWS_PER_TILE = NP // NS
RCHUNK = 40
NRCHUNK = ROWS_PER_TILE // RCHUNK
CHUNK_R = 80
SUPER_R = 2000

_vec_mesh = plsc.VectorSubcoreMesh(core_axis_name="c", subcore_axis_name="s")

_sc_params = pltpu.CompilerParams(needs_layout_passes=False,
                                  use_tc_tiling_on_sc=False)

_GATHER_DNUMS = lax.GatherDimensionNumbers(
    offset_dims=(), collapsed_slice_dims=(0,), start_index_map=(0,))


def _lane_gather(vec, idx):
    return lax.gather(vec, idx[:, None], _GATHER_DNUMS, slice_sizes=(1,),
                      mode=lax.GatherScatterMode.PROMISE_IN_BOUNDS)


def _node_body(x_ref, wn_ref, bn_ref, wq_ref, bq_ref, wk_ref, bk_ref,
               wv_ref, bv_ref, ws_ref, bs_ref, qd_ref, kv_ref, skip_ref):
    h = jnp.dot(x_ref[...], wn_ref[...], preferred_element_type=jnp.float32) + bn_ref[...]
    q = jnp.dot(h, wq_ref[...], preferred_element_type=jnp.float32) + bq_ref[...]
    qd_ref[...] = q * 0.25
    kv_ref[:, :HID] = jnp.dot(h, wk_ref[...], preferred_element_type=jnp.float32) + bk_ref[...]
    kv_ref[:, HID:] = jnp.dot(h, wv_ref[...], preferred_element_type=jnp.float32) + bv_ref[...]
    skip_ref[...] = jnp.dot(h, ws_ref[...], preferred_element_type=jnp.float32) + bs_ref[...]


def _edge_emb_body(ea_ref, we_ref, be_ref, e_ref):
    e_ref[...] = jnp.dot(ea_ref[...], we_ref[...], preferred_element_type=jnp.float32) + be_ref[...]


def _sc_edge_pass(qd_hbm, kv_hbm, e_hbm, ei_hbm, part_hbm,
                  srcA, srcB, dstA, dstB, qbuf, kvA, kvB, ebuf, msgbuf, acc,
                  semIA, semIB, semKA, semKB, semQ):
    cid = lax.axis_index("c")
    sid = lax.axis_index("s")
    wid = sid * NC + cid
    wbase = wid * EPW
    row0 = sid * ROWS_PER_TILE

    lane = lax.iota(jnp.int32, 16)
    idx15 = jnp.full((16,), 15, jnp.int32)

    @pl.loop(0, RCHUNK)
    def _zrow(i):
        for c in range(MSGW // 16):
            msgbuf[i, pl.ds(c * 16, 16)] = jnp.zeros((16,), jnp.float32)

    @pl.loop(0, NRCHUNK)
    def _zcopy(r):
        pltpu.sync_copy(msgbuf, acc.at[pl.ds(row0 + r * RCHUNK, RCHUNK)])

    plsc.subcore_barrier()

    NPAIR = EPW // (2 * CHUNK)

    def _idx_fire(base, sbuf, dbuf, semx):
        pltpu.async_copy(ei_hbm.at[0, pl.ds(base, CHUNK)], sbuf, semx)
        pltpu.async_copy(ei_hbm.at[1, pl.ds(base, CHUNK)], dbuf, semx)

    def _idx_drain(base, sbuf, dbuf, semx):
        pltpu.make_async_copy(ei_hbm.at[0, pl.ds(base, CHUNK)], sbuf, semx).wait()
        pltpu.make_async_copy(ei_hbm.at[1, pl.ds(base, CHUNK)], dbuf, semx).wait()

    def _compute(kvbuf, dbuf):
        @pl.loop(0, CHUNK, step=2)
        def _edge(i):
            for u in range(2):
                ii = i + u
                ke = []
                ve = []
                for h in range(HEADS):
                    sl = pl.ds(h * HD, HD)
                    kvv = kvbuf[ii, sl]
                    ev = ebuf[pl.ds(ii * HID + h * HD, HD)]
                    vv = kvbuf[ii, pl.ds(HID + h * HD, HD)]
                    ke.append(qbuf[ii, sl] * (kvv + ev))
                    ve.append(vv + ev)
                cs = [jnp.cumsum(t) for t in ke]
                sv = [_lane_gather(csx, idx15) for csx in cs]
                ex = [jnp.exp(s) for s in sv]
                w = [jnp.where(lane == h, ex[h], 0.0) for h in range(HEADS)]
                for h in range(HEADS):
                    msgbuf[ii, pl.ds(h * HD, HD)] = ex[h] * ve[h]
                w4 = [w[0] + w[1], w[2] + w[3], w[4] + w[5], w[6] + w[7]]
                msgbuf[ii, pl.ds(HID, 16)] = (w4[0] + w4[1]) + (w4[2] + w4[3])

        pltpu.sync_copy(msgbuf, acc.at[dbuf], add=True)

    _idx_fire(wbase, srcA, dstA, semIA)
    _idx_drain(wbase, srcA, dstA, semIA)
    pltpu.async_copy(kv_hbm.at[srcA], kvA, semKA)
    _idx_fire(wbase + CHUNK, srcB, dstB, semIB)

    @pl.loop(0, NPAIR)
    def _pair(p):
        baseA = pl.multiple_of(wbase + p * 2 * CHUNK, 8)
        baseB = pl.multiple_of(baseA + CHUNK, 8)
        pltpu.async_copy(qd_hbm.at[dstA], qbuf, semQ)
        pltpu.async_copy(e_hbm.at[pl.ds(baseA * HID, CHUNK * HID)], ebuf, semQ)
        _idx_drain(baseB, srcB, dstB, semIB)
        pltpu.async_copy(kv_hbm.at[srcB], kvB, semKB)
        pltpu.make_async_copy(kv_hbm.at[srcA], kvA, semKA).wait()
        pltpu.make_async_copy(qd_hbm.at[dstA], qbuf, semQ).wait()
        pltpu.make_async_copy(e_hbm.at[pl.ds(baseA * HID, CHUNK * HID)], ebuf, semQ).wait()
        _compute(kvA, dstA)

        @pl.when(p < NPAIR - 1)
        def _pfA():
            _idx_fire(baseA + 2 * CHUNK, srcA, dstA, semIA)

        pltpu.async_copy(qd_hbm.at[dstB], qbuf, semQ)
        pltpu.async_copy(e_hbm.at[pl.ds(baseB * HID, CHUNK * HID)], ebuf, semQ)
        pltpu.make_async_copy(kv_hbm.at[srcB], kvB, semKB).wait()
        pltpu.make_async_copy(qd_hbm.at[dstB], qbuf, semQ).wait()
        pltpu.make_async_copy(e_hbm.at[pl.ds(baseB * HID, CHUNK * HID)], ebuf, semQ).wait()
        _compute(kvB, dstB)

        @pl.when(p < NPAIR - 1)
        def _pfB():
            _idx_drain(baseA + 2 * CHUNK, srcA, dstA, semIA)
            pltpu.async_copy(kv_hbm.at[srcA], kvA, semKA)
            _idx_fire(baseB + 2 * CHUNK, srcB, dstB, semIB)

    plsc.subcore_barrier()

    @pl.loop(0, NRCHUNK)
    def _dump(r):
        rr = pl.multiple_of(row0 + r * RCHUNK, 8)
        pltpu.sync_copy(acc.at[pl.ds(rr, RCHUNK)], msgbuf)
        pltpu.sync_copy(msgbuf, part_hbm.at[cid, pl.ds(rr, RCHUNK)])


def _combine_body(p0_ref, p1_ref, skip_ref, wo_ref, bo_ref, p_ref):
    agg = p0_ref[0, :, :HID] + p1_ref[0, :, :HID]
    den = p0_ref[0, :, HID:HID + 16] + p1_ref[0, :, HID:HID + 16]
    j_iota = lax.broadcasted_iota(jnp.int32, (16, HID), 0)
    c_iota = lax.broadcasted_iota(jnp.int32, (16, HID), 1)
    expand = (c_iota // HD == j_iota).astype(jnp.float32)
    den_exp = jnp.dot(den, expand, preferred_element_type=jnp.float32)
    h2 = agg / (den_exp + 1e-16) + skip_ref[...]
    p_ref[...] = jnp.dot(h2, wo_ref[...], preferred_element_type=jnp.float32) + bo_ref[...]


def _sc_readout(p_hbm, ei_hbm, out_hbm, srcA, srcB, dstA, dstB,
                aA, bA, aB, bB, semIA, semIB, semA, semB):
    cid = lax.axis_index("c")
    sid = lax.axis_index("s")
    wid = sid * NC + cid
    wbase = wid * EPW
    NPAIR_R = EPW // (2 * CHUNK_R)

    def _ifire(base, sb, db, semx):
        pltpu.async_copy(ei_hbm.at[0, pl.ds(base, CHUNK_R)], sb, semx)
        pltpu.async_copy(ei_hbm.at[1, pl.ds(base, CHUNK_R)], db, semx)

    def _idrain(base, sb, db, semx):
        pltpu.make_async_copy(ei_hbm.at[0, pl.ds(base, CHUNK_R)], sb, semx).wait()
        pltpu.make_async_copy(ei_hbm.at[1, pl.ds(base, CHUNK_R)], db, semx).wait()

    def _gfire(sb, db, a, b, semx):
        pltpu.async_copy(p_hbm.at[sb], a, semx)
        pltpu.async_copy(p_hbm.at[db], b, semx)

    def _gdrain(sb, db, a, b, semx):
        pltpu.make_async_copy(p_hbm.at[sb], a, semx).wait()
        pltpu.make_async_copy(p_hbm.at[db], b, semx).wait()

    def _addout(a, b, base):
        @pl.loop(0, CHUNK_R, step=4)
        def _edge(i):
            for u in range(4):
                a[i + u, :] = a[i + u, :] + b[i + u, :]

        pltpu.sync_copy(a, out_hbm.at[pl.ds(base, CHUNK_R)])

    _ifire(wbase, srcA, dstA, semIA)
    _idrain(wbase, srcA, dstA, semIA)
    _gfire(srcA, dstA, aA, bA, semA)
    _ifire(wbase + CHUNK_R, srcB, dstB, semIB)

    @pl.loop(0, NPAIR_R)
    def _pair(p):
        baseA = pl.multiple_of(wbase + p * 2 * CHUNK_R, 8)
        baseB = pl.multiple_of(baseA + CHUNK_R, 8)
        _idrain(baseB, srcB, dstB, semIB)
        _gfire(srcB, dstB, aB, bB, semB)
        _gdrain(srcA, dstA, aA, bA, semA)
        _addout(aA, bA, baseA)
        _ifire(baseA + 2 * CHUNK_R, srcA, dstA, semIA)
        _gdrain(srcB, dstB, aB, bB, semB)
        _addout(aB, bB, baseB)
        _idrain(baseA + 2 * CHUNK_R, srcA, dstA, semIA)
        _gfire(srcA, dstA, aA, bA, semA)

        @pl.when(p < NPAIR_R - 1)
        def _pf():
            _ifire(baseB + 2 * CHUNK_R, srcB, dstB, semIB)

    _gdrain(srcA, dstA, aA, bA, semA)
    _addout(aA, bA, wbase + 2 * NPAIR_R * CHUNK_R)


def kernel(x, edge_index, edge_attr, W_node, b_node, W_edge, b_edge, W_q, b_q,
           W_k, b_k, W_v, b_v, W_skip, b_skip, W_out, b_out):
    B1 = 1000
    qd, kv, skip = pl.pallas_call(
        _node_body,
        grid=(N // B1,),
        in_specs=[pl.BlockSpec((B1, HID), lambda i: (i, 0))]
        + [pl.BlockSpec((HID, HID), lambda i: (0, 0)), pl.BlockSpec((1, HID), lambda i: (0, 0))] * 5,
        out_specs=[
            pl.BlockSpec((B1, HID), lambda i: (i, 0)),
            pl.BlockSpec((B1, 2 * HID), lambda i: (i, 0)),
            pl.BlockSpec((B1, HID), lambda i: (i, 0)),
        ],
        out_shape=[
            jax.ShapeDtypeStruct((N, HID), jnp.float32),
            jax.ShapeDtypeStruct((N, 2 * HID), jnp.float32),
            jax.ShapeDtypeStruct((N, HID), jnp.float32),
        ],
    )(x, W_node.T, b_node[None, :], W_q.T, b_q[None, :], W_k.T, b_k[None, :],
      W_v.T, b_v[None, :], W_skip.T, b_skip[None, :])

    BE = 8000
    e = pl.pallas_call(
        _edge_emb_body,
        grid=(E // BE,),
        in_specs=[
            pl.BlockSpec((BE, 16), lambda i: (i, 0)),
            pl.BlockSpec((16, HID), lambda i: (0, 0)),
            pl.BlockSpec((1, HID), lambda i: (0, 0)),
        ],
        out_specs=pl.BlockSpec((BE, HID), lambda i: (i, 0)),
        out_shape=jax.ShapeDtypeStruct((E, HID), jnp.float32),
    )(edge_attr, W_edge.T, b_edge[None, :])

    part = pl.kernel(
        _sc_edge_pass,
        out_type=jax.ShapeDtypeStruct((NC, NP, MSGW), jnp.float32),
        mesh=_vec_mesh,
        scratch_types=[
            pltpu.VMEM((CHUNK,), jnp.int32),
            pltpu.VMEM((CHUNK,), jnp.int32),
            pltpu.VMEM((CHUNK,), jnp.int32),
            pltpu.VMEM((CHUNK,), jnp.int32),
            pltpu.VMEM((CHUNK, HID), jnp.float32),
            pltpu.VMEM((CHUNK, 2 * HID), jnp.float32),
            pltpu.VMEM((CHUNK, 2 * HID), jnp.float32),
            pltpu.VMEM((CHUNK * HID,), jnp.float32),
            pltpu.VMEM((CHUNK, MSGW), jnp.float32),
            pltpu.VMEM_SHARED((NP, MSGW), jnp.float32),
            pltpu.SemaphoreType.DMA,
            pltpu.SemaphoreType.DMA,
            pltpu.SemaphoreType.DMA,
            pltpu.SemaphoreType.DMA,
            pltpu.SemaphoreType.DMA,
        ],
        compiler_params=_sc_params,
    )(qd, kv, e.reshape(-1), edge_index)

    B3 = 1000
    P = pl.pallas_call(
        _combine_body,
        grid=(N // B3,),
        in_specs=[
            pl.BlockSpec((1, B3, MSGW), lambda i: (0, i, 0)),
            pl.BlockSpec((1, B3, MSGW), lambda i: (1, i, 0)),
            pl.BlockSpec((B3, HID), lambda i: (i, 0)),
            pl.BlockSpec((HID, NUM_CLASSES), lambda i: (0, 0)),
            pl.BlockSpec((1, NUM_CLASSES), lambda i: (0, 0)),
        ],
        out_specs=pl.BlockSpec((B3, NUM_CLASSES), lambda i: (i, 0)),
        out_shape=jax.ShapeDtypeStruct((N, NUM_CLASSES), jnp.float32),
    )(part, part, skip, W_out.T, 0.5 * b_out[None, :])

    edge_preds = pl.kernel(
        _sc_readout,
        out_type=jax.ShapeDtypeStruct((E, NUM_CLASSES), jnp.float32),
        mesh=_vec_mesh,
        scratch_types=[
            pltpu.VMEM((CHUNK_R,), jnp.int32),
            pltpu.VMEM((CHUNK_R,), jnp.int32),
            pltpu.VMEM((CHUNK_R,), jnp.int32),
            pltpu.VMEM((CHUNK_R,), jnp.int32),
            pltpu.VMEM((CHUNK_R, NUM_CLASSES), jnp.float32),
            pltpu.VMEM((CHUNK_R, NUM_CLASSES), jnp.float32),
            pltpu.VMEM((CHUNK_R, NUM_CLASSES), jnp.float32),
            pltpu.VMEM((CHUNK_R, NUM_CLASSES), jnp.float32),
            pltpu.SemaphoreType.DMA,
            pltpu.SemaphoreType.DMA,
            pltpu.SemaphoreType.DMA,
            pltpu.SemaphoreType.DMA,
        ],
        compiler_params=_sc_params,
    )(P, edge_index)

    return edge_preds
